# baseline (speedup 1.0000x reference)
import functools

import jax
import jax.numpy as jnp
from jax import lax
from jax.experimental import pallas as pl
from jax.experimental.pallas import tpu as pltpu
from jax.experimental.pallas import tpu_sc as plsc

NC = 2
NS = 16
L = 16
NW = NC * NS
BW = 128
PACK = 256
SPLIT = 500224


def _pack_table(tok_t, V, D):
    CW = 512
    nblk = SPLIT // (CW // 2)

    def body(a_ref, o_ref):
        y = jnp.transpose(a_ref[...])
        o_ref[...] = jnp.concatenate([y[:CW // 2], y[CW // 2:]], axis=1)

    return pl.pallas_call(
        body,
        grid=(nblk,),
        in_specs=[pl.BlockSpec((D, CW), lambda j: (0, j))],
        out_specs=pl.BlockSpec((CW // 2, 2 * D), lambda j: (j, 0)),
        out_shape=jax.ShapeDtypeStruct((SPLIT, 2 * D), jnp.float32),
    )(tok_t)


@functools.cache
def _gather_call(B, T, D, V):
    jn = B // BW
    nstrip = T * jn
    strips_w = nstrip // NW

    mesh = plsc.VectorSubcoreMesh(core_axis_name="c", subcore_axis_name="s")

    @functools.partial(
        pl.kernel,
        out_type=jax.ShapeDtypeStruct((B // 8, T // 2, 8, 2 * D), jnp.float32),
        mesh=mesh,
        compiler_params=pltpu.CompilerParams(use_tc_tiling_on_sc=False),
        scratch_types=[
            pltpu.VMEM((2, BW), jnp.int32),
            pltpu.VMEM((2 * BW, D), jnp.float32),
            pltpu.VMEM((BW // 8, 8, 2 * D), jnp.float32),
            pltpu.SemaphoreType.DMA,
        ],
    )
    def emb(row_hbm, tok_hbm, out_hbm, row_v, g0, s0, sg0):
        wid = lax.axis_index("s") * NC + lax.axis_index("c")
        npair = strips_w // 2
        j = (wid * npair) // (T // 2)
        tp0 = (wid * npair) % (T // 2)

        def pair_step(p, carry):
            tp = tp0 + p
            cps = []
            for h in range(2):
                pltpu.sync_copy(
                    row_hbm.at[2 * tp + h, pl.ds(j * BW, BW)], row_v.at[h])
                cp = pltpu.make_async_copy(
                    tok_hbm.at[row_v.at[h]],
                    g0.at[pl.ds(h * BW, BW)], sg0)
                cp.start()
                cps.append(cp)
            for cp in cps:
                cp.wait()

            def e_body(e, carry2):
                er, ed = lax.shift_right_logical(e, 3), e & 7
                for h in range(2):
                    for q in range(D // L):
                        s0[er, ed, pl.ds(h * D + q * L, L)] = (
                            g0[h * BW + e, pl.ds(q * L, L)])
                return carry2

            lax.fori_loop(0, BW, e_body, 0)
            pltpu.sync_copy(
                s0, out_hbm.at[pl.ds(j * (BW // 8), BW // 8), tp, :, :])
            return carry

        lax.fori_loop(0, npair, pair_step, 0)

    return emb


def _format_out(mid, pos3, B, T, D):
    def body(x_ref, p_ref, o_ref):
        x = x_ref[...]
        p = p_ref[0]
        for h in range(2):
            y = jnp.transpose(x[:, h * D:(h + 1) * D]) + p[:, h:h + 1]
            o_ref[h, :, 0, :, :] = y.reshape(D // 8, 8, BW)

    return pl.pallas_call(
        body,
        grid=(T // 2, B // BW),
        in_specs=[
            pl.BlockSpec((BW, 2 * D), lambda tp, j: (j, tp)),
            pl.BlockSpec((1, D, 2), lambda tp, j: (tp, 0, 0)),
        ],
        out_specs=pl.BlockSpec((2, D // 8, 1, 8, BW),
                               lambda tp, j: (tp, 0, j, 0, 0)),
        out_shape=jax.ShapeDtypeStruct((T, D // 8, B // BW, 8, BW),
                                       jnp.float32),
    )(mid, pos3)


def kernel(idx, tok_table, pos_table):
    B, T = idx.shape
    V, D = tok_table.shape
    assert B % BW == 0 and T % 2 == 0 and D == 64 and V <= 2 * SPLIT
    tok2 = _pack_table(tok_table.T, V, D)
    tok_lin = tok2.reshape(2 * SPLIT, D)
    idx_t = idx.astype(jnp.int32).T
    rows = (512 * (idx_t >> 9) + 2 * (idx_t & 255) + ((idx_t >> 8) & 1))
    out4 = _gather_call(B, T, D, V)(rows, tok_lin)
    mid = out4.transpose(0, 2, 1, 3).reshape(B, T * D)
    pos3 = pos_table.reshape(T // 2, 2, D).transpose(0, 2, 1)
    out5 = _format_out(mid, pos3, B, T, D)
    return out5.transpose(2, 4, 0, 1, 3).reshape(B, T, D)


# speedup vs baseline: 2.3437x; 2.3437x over previous
import functools

import jax
import jax.numpy as jnp
from jax import lax
from jax.experimental import pallas as pl
from jax.experimental.pallas import tpu as pltpu
from jax.experimental.pallas import tpu_sc as plsc

NC = 2
NS = 16
L = 16
NW = NC * NS
HALF = 100


@functools.cache
def _emb_call(B, T, D, V):
    seq_w = B // NW
    vpr = D // L

    mesh = plsc.VectorSubcoreMesh(core_axis_name="c", subcore_axis_name="s")

    @functools.partial(
        pl.kernel,
        out_type=jax.ShapeDtypeStruct((B, T, D), jnp.float32),
        mesh=mesh,
        compiler_params=pltpu.CompilerParams(use_tc_tiling_on_sc=False),
        scratch_types=[
            pltpu.VMEM((seq_w * 2, HALF), jnp.int32),
            pltpu.VMEM((T, D), jnp.float32),
            pltpu.VMEM((T, D), jnp.float32),
            pltpu.VMEM((T, D), jnp.float32),
            pltpu.VMEM((T, D), jnp.float32),
            pltpu.VMEM((T, D), jnp.float32),
            pltpu.SemaphoreType.DMA,
            pltpu.SemaphoreType.DMA,
            pltpu.SemaphoreType.DMA,
            pltpu.SemaphoreType.DMA,
        ],
    )
    def emb(idx_hbm, tok_hbm, pos_hbm, out_hbm, idx_v, pos_v,
            g0, g1, o0, o1, sg0, sg1, so0, so1):
        wid = lax.axis_index("s") * NC + lax.axis_index("c")
        pltpu.sync_copy(idx_hbm.at[wid], idx_v)
        pltpu.sync_copy(pos_hbm, pos_v)

        gbuf, obuf = (g0, g1), (o0, o1)
        gsem, osem = (sg0, sg1), (so0, so1)

        def fire_gather(j):
            b = j % 2
            return [
                pltpu.make_async_copy(
                    tok_hbm.at[idx_v.at[2 * j + h]],
                    gbuf[b].at[pl.ds(h * HALF, HALF)],
                    gsem[b],
                ) for h in range(2)
            ]
        for cp in [c for j in range(2) for c in fire_gather(j)]:
            cp.start()

        gh = {0: fire_gather(0), 1: fire_gather(1)}
        oh = {}
        for j in range(seq_w):
            b = j % 2
            for cp in gh[j]:
                cp.wait()
            if j >= 2:
                oh[j - 2].wait()

            def row_body(r, carry, _g=gbuf[b], _o=obuf[b]):
                for q in range(vpr):
                    s = pl.ds(q * L, L)
                    _o[r, s] = _g[r, s] + pos_v[r, s]
                return carry

            lax.fori_loop(0, T, row_body, 0)

            oh[j] = pltpu.make_async_copy(
                obuf[b], out_hbm.at[wid * seq_w + j], osem[b])
            oh[j].start()
            if j + 2 < seq_w:
                gh[j + 2] = fire_gather(j + 2)
                for cp in gh[j + 2]:
                    cp.start()
        oh[seq_w - 2].wait()
        oh[seq_w - 1].wait()

    return emb


def kernel(idx, tok_table, pos_table):
    B, T = idx.shape
    V, D = tok_table.shape
    assert B % NW == 0 and T == 2 * HALF and D % L == 0
    idx3 = idx.astype(jnp.int32).reshape(NW, (B // NW) * 2, HALF)
    return _emb_call(B, T, D, V)(idx3, tok_table, pos_table)
